# baseline (speedup 1.0000x reference)
import jax
import jax.numpy as jnp
import numpy as np
from jax import lax
from jax.experimental import pallas as pl
from jax.experimental.pallas import tpu as pltpu

_N = 16384
_C = 1000
_K = _N // 2
_R = 512
_G = _N // _R
_S = 4
_H = _G // _S

def _body(p0, p1, p2, p3, out_ref, loss_ref):
    i = pl.program_id(0)
    lane = lax.broadcasted_iota(jnp.int32, (_R, _G), 1)
    cur = loss_ref[...]
    for s, p in enumerate((p0, p1, p2, p3)):
        cur = jnp.where(lane == i + s * _H, p[:, 0:1], cur)
    loss_ref[...] = cur

    @pl.when(i == _H - 1)
    def _sel():
        out_ref[...] = jnp.sum(loss_ref[...]).reshape(1, 1)

def kernel(pred, target):
    mk = lambda s: pl.BlockSpec((_R, _C), lambda i, s=s: (i + s * _H, 0))
    out = pl.pallas_call(
        _body,
        grid=(_H,),
        in_specs=[mk(0), mk(1), mk(2), mk(3)],
        out_specs=pl.BlockSpec((1, 1), lambda i: (0, 0)),
        out_shape=jax.ShapeDtypeStruct((1, 1), jnp.float32),
        scratch_shapes=[pltpu.VMEM((_R, _G), jnp.float32)],
        compiler_params=pltpu.CompilerParams(dimension_semantics=("arbitrary",)),
    )(pred, pred, pred, pred)
    return out[0, 0]
